# Initial kernel scaffold; baseline (speedup 1.0000x reference)
import jax
import jax.numpy as jnp
from jax.experimental import pallas as pl


def kernel(input_tensor, src_table, pos_table):
    raise NotImplementedError("write your pallas kernel here")


# speedup vs baseline: 2.2037x; 2.2037x over previous
import functools

import jax
import jax.numpy as jnp
from jax import lax
from jax.experimental import pallas as pl
from jax.experimental.pallas import tpu as pltpu
from jax.experimental.pallas import tpu_sc as plsc

EMBED = 64
SEQ = 200
BATCH = 4096
ROWS = BATCH * SEQ
NC, NS = 2, 16
NW = NC * NS
ROWS_PER_W = ROWS // NW
SEQS_PER_W = ROWS_PER_W // SEQ
SCALE = 8.0
GA = 128
GB = SEQ - GA


def _sc_embed(idx_flat, table, pos):
  mesh = plsc.VectorSubcoreMesh(core_axis_name="c", subcore_axis_name="s")

  @functools.partial(
      pl.kernel,
      mesh=mesh,
      compiler_params=pltpu.CompilerParams(use_tc_tiling_on_sc=False),
      out_type=jax.ShapeDtypeStruct((ROWS, EMBED), jnp.float32),
      scratch_types=[
          pltpu.VMEM((GA,), jnp.int32),
          pltpu.VMEM((GB,), jnp.int32),
          pltpu.VMEM((SEQ, EMBED), jnp.float32),
          pltpu.VMEM((SEQ, EMBED), jnp.float32),
          pltpu.SemaphoreType.DMA,
      ],
  )
  def k(idx_hbm, table_hbm, pos_hbm, out_hbm, idx_a, idx_b, rows_v, pos_v, sem):
    wid = lax.axis_index("s") * NC + lax.axis_index("c")
    base = wid * ROWS_PER_W
    pltpu.sync_copy(pos_hbm.at[pl.ds(0, SEQ)], pos_v)

    def seq_body(c, carry):
      row0 = base + c * SEQ
      pltpu.sync_copy(idx_hbm.at[pl.ds(row0, GA)], idx_a)
      pltpu.sync_copy(idx_hbm.at[pl.ds(row0 + GA, GB)], idx_b)
      cp_a = pltpu.async_copy(table_hbm.at[idx_a], rows_v.at[pl.ds(0, GA)], sem)
      cp_b = pltpu.async_copy(table_hbm.at[idx_b], rows_v.at[pl.ds(GA, GB)], sem)
      cp_a.wait()
      cp_b.wait()

      def row_body(r, rc):
        for q in range(EMBED // 16):
          sl = pl.ds(q * 16, 16)
          rows_v[r, sl] = rows_v[r, sl] * SCALE + pos_v[r, sl]
        return rc
      lax.fori_loop(0, SEQ, row_body, 0, unroll=4)

      pltpu.sync_copy(rows_v, out_hbm.at[pl.ds(row0, SEQ)])
      return carry

    lax.fori_loop(0, SEQS_PER_W, seq_body, 0)

  return k(idx_flat, table, pos)


def kernel(input_tensor, src_table, pos_table):
  idx_flat = input_tensor.reshape(ROWS).astype(jnp.int32)
  out = _sc_embed(idx_flat, src_table, pos_table)
  return out.reshape(BATCH, SEQ, EMBED)


# speedup vs baseline: 2.8982x; 1.3151x over previous
import functools

import jax
import jax.numpy as jnp
from jax import lax
from jax.experimental import pallas as pl
from jax.experimental.pallas import tpu as pltpu
from jax.experimental.pallas import tpu_sc as plsc

EMBED = 64
SEQ = 200
BATCH = 4096
ROWS = BATCH * SEQ
NC, NS = 2, 16
NW = NC * NS
ROWS_PER_W = ROWS // NW
SEQS_PER_W = ROWS_PER_W // SEQ
SCALE = 8.0
GA = 128
GB = SEQ - GA
NBUF = 4


def _sc_embed(idx_flat, table, pos):
  mesh = plsc.VectorSubcoreMesh(core_axis_name="c", subcore_axis_name="s")

  @functools.partial(
      pl.kernel,
      mesh=mesh,
      compiler_params=pltpu.CompilerParams(use_tc_tiling_on_sc=False),
      out_type=jax.ShapeDtypeStruct((ROWS, EMBED), jnp.float32),
      scratch_types=[
          pltpu.VMEM((ROWS_PER_W,), jnp.int32),
          pltpu.VMEM((SEQ, EMBED), jnp.float32),
          [pltpu.VMEM((SEQ, EMBED), jnp.float32)] * NBUF,
          [pltpu.SemaphoreType.DMA] * NBUF,
          [pltpu.SemaphoreType.DMA] * NBUF,
      ],
  )
  def k(idx_hbm, table_hbm, pos_hbm, out_hbm, idx_all, pos_v, bufs, gsem, ssem):
    wid = lax.axis_index("s") * NC + lax.axis_index("c")
    base = wid * ROWS_PER_W
    pltpu.sync_copy(idx_hbm.at[pl.ds(base, ROWS_PER_W)], idx_all)
    pltpu.sync_copy(pos_hbm.at[pl.ds(0, SEQ)], pos_v)

    def start_gather(c, b):
      off = c * SEQ
      pltpu.async_copy(table_hbm.at[idx_all.at[pl.ds(off, GA)]],
                       bufs[b].at[pl.ds(0, GA)], gsem[b])
      pltpu.async_copy(table_hbm.at[idx_all.at[pl.ds(off + GA, GB)]],
                       bufs[b].at[pl.ds(GA, GB)], gsem[b])

    def wait_gather(b):
      pltpu.make_async_copy(table_hbm.at[idx_all.at[pl.ds(0, GA)]],
                            bufs[b].at[pl.ds(0, GA)], gsem[b]).wait()
      pltpu.make_async_copy(table_hbm.at[idx_all.at[pl.ds(0, GB)]],
                            bufs[b].at[pl.ds(GA, GB)], gsem[b]).wait()

    def start_scatter(c, b):
      pltpu.async_copy(bufs[b], out_hbm.at[pl.ds(base + c * SEQ, SEQ)], ssem[b])

    def wait_scatter(b):
      pltpu.make_async_copy(bufs[b], out_hbm.at[pl.ds(base, SEQ)],
                            ssem[b]).wait()

    def compute(b):
      buf = bufs[b]

      def row_body(r, rc):
        for q in range(EMBED // 16):
          sl = pl.ds(q * 16, 16)
          buf[r, sl] = buf[r, sl] * SCALE + pos_v[r, sl]
        return rc

      lax.fori_loop(0, SEQ, row_body, 0, unroll=4)

    start_gather(0, 0)
    start_gather(1, 1)

    def step(i, carry):
      for b in range(NBUF):
        c = i * NBUF + b
        wait_gather(b)
        compute(b)
        nb = (b + 2) % NBUF

        @pl.when(c >= 2)
        def _():
          wait_scatter(nb)

        @pl.when(c + 2 < SEQS_PER_W)
        def _():
          start_gather(c + 2, nb)

        start_scatter(c, b)
      return carry

    lax.fori_loop(0, SEQS_PER_W // NBUF, step, 0)
    wait_scatter((SEQS_PER_W - 2) % NBUF)
    wait_scatter((SEQS_PER_W - 1) % NBUF)

  return k(idx_flat, table, pos)


def kernel(input_tensor, src_table, pos_table):
  idx_flat = input_tensor.reshape(ROWS).astype(jnp.int32)
  out = _sc_embed(idx_flat, src_table, pos_table)
  return out.reshape(BATCH, SEQ, EMBED)
